# baseline (speedup 1.0000x reference)
import jax
import jax.numpy as jnp
from jax.experimental import pallas as pl
from jax.experimental.pallas import tpu as pltpu

N = 10000
D_IN = 128
D_OUT = 128
BM = 400
NBUF = 3
NSTEPS = N // BM


def _gcn_kernel(x_ref, w_ref, b_ref, adj_hbm, out_hbm, h_ref):
    h_ref[...] = jax.lax.dot_general(
        x_ref[...], w_ref[...],
        (((1,), (1,)), ((), ())),
        preferred_element_type=jnp.float32,
    ) + b_ref[...]

    def inner(adj_blk, out_blk):
        out_blk[...] = jnp.dot(
            adj_blk[...], h_ref[...], preferred_element_type=jnp.float32
        )

    pltpu.emit_pipeline(
        inner,
        grid=(NSTEPS,),
        in_specs=[
            pl.BlockSpec((BM, N), lambda i: (i, 0),
                         pipeline_mode=pl.Buffered(buffer_count=NBUF)),
        ],
        out_specs=[pl.BlockSpec((BM, D_OUT), lambda i: (i, 0))],
    )(adj_hbm, out_hbm)


@jax.jit
def kernel(x, adj, W, b):
    out = pl.pallas_call(
        _gcn_kernel,
        in_specs=[
            pl.BlockSpec((N, D_IN), lambda: (0, 0)),
            pl.BlockSpec((D_OUT, D_IN), lambda: (0, 0)),
            pl.BlockSpec((1, D_OUT), lambda: (0, 0)),
            pl.BlockSpec(memory_space=pl.ANY),
        ],
        out_specs=pl.BlockSpec(memory_space=pl.ANY),
        out_shape=jax.ShapeDtypeStruct((N, D_OUT), jnp.float32),
        scratch_shapes=[pltpu.VMEM((N, D_OUT), jnp.float32)],
        compiler_params=pltpu.CompilerParams(vmem_limit_bytes=100 * 1024 * 1024),
    )(x, W, b.reshape(1, D_OUT), adj)
    return out


# speedup vs baseline: 1.0435x; 1.0435x over previous
import jax
import jax.numpy as jnp
from jax.experimental import pallas as pl
from jax.experimental.pallas import tpu as pltpu

N = 10000
D_IN = 128
D_OUT = 128
BM = 400


def _gcn_kernel(x_ref, w_ref, b_ref, adj_ref, out_ref, h_ref):
    @pl.when(pl.program_id(0) == 0)
    def _():
        h_ref[...] = jax.lax.dot_general(
            x_ref[...], w_ref[...],
            (((1,), (1,)), ((), ())),
            preferred_element_type=jnp.float32,
        ) + b_ref[...]

    out_ref[...] = jnp.dot(
        adj_ref[...], h_ref[...], preferred_element_type=jnp.float32
    )


@jax.jit
def kernel(x, adj, W, b):
    out = pl.pallas_call(
        _gcn_kernel,
        grid=(N // BM,),
        in_specs=[
            pl.BlockSpec((N, D_IN), lambda i: (0, 0)),
            pl.BlockSpec((D_OUT, D_IN), lambda i: (0, 0)),
            pl.BlockSpec((1, D_OUT), lambda i: (0, 0)),
            pl.BlockSpec((BM, N), lambda i: (i, 0)),
        ],
        out_specs=pl.BlockSpec((BM, D_OUT), lambda i: (i, 0)),
        out_shape=jax.ShapeDtypeStruct((N, D_OUT), jnp.float32),
        scratch_shapes=[pltpu.VMEM((N, D_OUT), jnp.float32)],
    )(x, W, b.reshape(1, D_OUT), adj)
    return out
